# baseline (speedup 1.0000x reference)
import functools

import jax
import jax.numpy as jnp
from jax import lax
from jax.experimental import pallas as pl
from jax.experimental.pallas import tpu as pltpu
from jax.experimental.pallas import tpu_sc as plsc

N = 10000
E = 320000
D = 128
H = D // 2
NC = 2
NS = 16
EPW = E // NS
C = 80
NCHUNK = EPW // C
NBUF = 5
LAG = NBUF // 2
NITER = NCHUNK // NBUF
RPS = 624
TAIL = N - NS * RPS


def _sc_body(with_deg, *refs):
    if with_deg:
        (table_hbm, src_hbm, dst_hbm, zrow_hbm, zdeg_hbm, ones_hbm,
         sum_hbm, deg_hbm,
         idx_v, dst_v, rows_v, ones_v, acc_sh, deg_sh, gsem, ssem, dsem) = refs
    else:
        (table_hbm, src_hbm, dst_hbm, zrow_hbm,
         sum_hbm,
         idx_v, dst_v, rows_v, acc_sh, gsem, ssem) = refs

    cid = lax.axis_index("c")
    sid = lax.axis_index("s")
    last = sid == NS - 1

    pltpu.sync_copy(src_hbm.at[cid * NS + sid], idx_v)
    pltpu.sync_copy(dst_hbm.at[sid], dst_v)
    for b in range(LAG):
        pltpu.async_copy(table_hbm.at[idx_v.at[b]], rows_v.at[b], gsem)

    base = pl.multiple_of(sid * RPS, 8)
    pltpu.sync_copy(zrow_hbm, acc_sh.at[pl.ds(base, RPS)])

    @pl.when(last)
    def _():
        pltpu.sync_copy(zrow_hbm.at[pl.ds(0, TAIL)], acc_sh.at[pl.ds(NS * RPS, TAIL)])

    if with_deg:
        pltpu.sync_copy(ones_hbm, ones_v)
        pltpu.sync_copy(zdeg_hbm, deg_sh.at[pl.ds(base, RPS)])

        @pl.when(last)
        def _():
            pltpu.sync_copy(zdeg_hbm.at[pl.ds(0, TAIL)],
                            deg_sh.at[pl.ds(NS * RPS, TAIL)])

    plsc.subcore_barrier()

    def giter(g, carry):
        jbase = g * NBUF
        for b in range(NBUF):
            j = jbase + b
            pltpu.make_async_copy(table_hbm.at[idx_v.at[j]],
                                  rows_v.at[b], gsem).wait()
            pltpu.async_copy(rows_v.at[b], acc_sh.at[dst_v.at[j]], ssem, add=True)
            if with_deg:
                @pl.when(lax.rem(j, NC) == cid)
                def _():
                    pltpu.async_copy(ones_v, deg_sh.at[dst_v.at[j]], dsem,
                                     add=True)

            @pl.when(j >= LAG)
            def _():
                pltpu.make_async_copy(rows_v.at[(b + NBUF - LAG) % NBUF],
                                      acc_sh.at[dst_v.at[j]], ssem).wait()

            @pl.when(j + LAG < NCHUNK)
            def _():
                pltpu.async_copy(table_hbm.at[idx_v.at[j + LAG]],
                                 rows_v.at[(b + LAG) % NBUF], gsem)
        return carry

    lax.fori_loop(0, NITER, giter, 0)

    for b in range(LAG):
        pltpu.make_async_copy(rows_v.at[b], acc_sh.at[dst_v.at[0]], ssem).wait()
    if with_deg:
        def ddrain(i, carry):
            pltpu.make_async_copy(ones_v, deg_sh.at[dst_v.at[0]], dsem).wait()
            return carry

        lax.fori_loop(0, NCHUNK // NC, ddrain, 0)

    plsc.subcore_barrier()

    pltpu.sync_copy(acc_sh.at[pl.ds(base, RPS)], sum_hbm.at[cid, pl.ds(base, RPS)])

    @pl.when(last)
    def _():
        pltpu.sync_copy(acc_sh.at[pl.ds(NS * RPS, TAIL)],
                        sum_hbm.at[cid, pl.ds(NS * RPS, TAIL)])

    if with_deg:
        pltpu.sync_copy(deg_sh.at[pl.ds(base, RPS)], deg_hbm.at[cid, pl.ds(base, RPS)])

        @pl.when(last)
        def _():
            pltpu.sync_copy(deg_sh.at[pl.ds(NS * RPS, TAIL)],
                            deg_hbm.at[cid, pl.ds(NS * RPS, TAIL)])


def _make_sc_scatter(with_deg):
    out_type = [jax.ShapeDtypeStruct((NC, N, H), jnp.float32)]
    if with_deg:
        out_type.append(jax.ShapeDtypeStruct((NC, N, 16), jnp.float32))
    scratch = [
        pltpu.VMEM((NCHUNK, C), jnp.int32),
        pltpu.VMEM((NCHUNK, C), jnp.int32),
        pltpu.VMEM((NBUF, C, H), jnp.float32),
    ]
    if with_deg:
        scratch.append(pltpu.VMEM((C, 16), jnp.float32))
    scratch.append(pltpu.VMEM_SHARED((N, H), jnp.float32))
    if with_deg:
        scratch.append(pltpu.VMEM_SHARED((N, 16), jnp.float32))
    scratch.append(pltpu.SemaphoreType.DMA)
    scratch.append(pltpu.SemaphoreType.DMA)
    if with_deg:
        scratch.append(pltpu.SemaphoreType.DMA)

    mesh = plsc.VectorSubcoreMesh(core_axis_name="c", subcore_axis_name="s")
    return pl.kernel(
        functools.partial(_sc_body, with_deg),
        out_type=out_type,
        mesh=mesh,
        scratch_types=scratch,
        compiler_params=pltpu.CompilerParams(use_tc_tiling_on_sc=False),
    )


_sc_scatter_deg = _make_sc_scatter(True)
_sc_scatter = _make_sc_scatter(False)


def _tc_layer_body(relu, s_ref, d_ref, x_ref, wl_ref, wr_ref, b_ref, out_ref):
    deg = jnp.maximum(d_ref[0, :, 0:1] + d_ref[1, :, 0:1], 1.0)
    inv = 1.0 / deg
    z = (jnp.dot(s_ref[0] * inv, wl_ref[0:H, :], preferred_element_type=jnp.float32)
         + jnp.dot(s_ref[1] * inv, wl_ref[H:D, :], preferred_element_type=jnp.float32)
         + jnp.dot(x_ref[...], wr_ref[...], preferred_element_type=jnp.float32)
         + b_ref[...])
    out_ref[...] = jnp.maximum(z, 0.0) if relu else z


_TC_BLOCK = 1000


def _tc_layer(relu, s, d, x, wl_t, wr_t, b):
    grid = (N // _TC_BLOCK,)
    return pl.pallas_call(
        functools.partial(_tc_layer_body, relu),
        grid=grid,
        in_specs=[
            pl.BlockSpec((NC, _TC_BLOCK, H), lambda i: (0, i, 0)),
            pl.BlockSpec((NC, _TC_BLOCK, 16), lambda i: (0, i, 0)),
            pl.BlockSpec((_TC_BLOCK, D), lambda i: (i, 0)),
            pl.BlockSpec((D, D), lambda i: (0, 0)),
            pl.BlockSpec((D, D), lambda i: (0, 0)),
            pl.BlockSpec((1, D), lambda i: (0, 0)),
        ],
        out_specs=pl.BlockSpec((_TC_BLOCK, D), lambda i: (i, 0)),
        out_shape=jax.ShapeDtypeStruct((N, D), jnp.float32),
    )(s, d, x, wl_t, wr_t, b)


def kernel(x, edge_index, W1_l, W1_r, b1, W2_l, W2_r, b2):
    src = edge_index[0].astype(jnp.int32).reshape(NS, NCHUNK, C)
    dst = edge_index[1].astype(jnp.int32).reshape(NS, NCHUNK, C)
    src2 = jnp.concatenate([2 * src, 2 * src + 1], axis=0)
    zrow = jnp.zeros((RPS, H), jnp.float32)
    zdeg = jnp.zeros((RPS, 16), jnp.float32)
    ones = jnp.ones((C, 16), jnp.float32)

    s1, deg = _sc_scatter_deg(x.reshape(2 * N, H), src2, dst, zrow, zdeg, ones)
    h = _tc_layer(True, s1, deg, x, W1_l.T, W1_r.T, b1.reshape(1, D))
    (s2,) = _sc_scatter(h.reshape(2 * N, H), src2, dst, zrow)
    out = _tc_layer(False, s2, deg, h, W2_l.T, W2_r.T, b2.reshape(1, D))
    return out


# speedup vs baseline: 1.1061x; 1.1061x over previous
import functools

import jax
import jax.numpy as jnp
from jax import lax
from jax.experimental import pallas as pl
from jax.experimental.pallas import tpu as pltpu
from jax.experimental.pallas import tpu_sc as plsc

N = 10000
E = 320000
D = 128
H = D // 2
NC = 2
NS = 16
EPW = E // NS
C = 80
NCHUNK = EPW // C
NBUF = 5
LAG = NBUF // 2
NITER = NCHUNK // NBUF
RPS = 624
TAIL = N - NS * RPS


def _sc_body(with_deg, *refs):
    if with_deg:
        (table_hbm, src_hbm, dst_hbm, zrow_hbm, zdeg_hbm, ones_hbm,
         sum_hbm, deg_hbm,
         idx_v, dst_v, rows_v, ones_v, acc_sh, deg_sh, gsem, ssem, dsem) = refs
    else:
        (table_hbm, src_hbm, dst_hbm, zrow_hbm,
         sum_hbm,
         idx_v, dst_v, rows_v, acc_sh, gsem, ssem) = refs

    cid = lax.axis_index("c")
    sid = lax.axis_index("s")
    last = sid == NS - 1

    pltpu.sync_copy(src_hbm.at[cid * NS + sid], idx_v)
    pltpu.sync_copy(dst_hbm.at[sid], dst_v)
    for b in range(NBUF):
        pltpu.async_copy(table_hbm.at[idx_v.at[b]], rows_v.at[b], gsem)

    base = pl.multiple_of(sid * RPS, 8)
    pltpu.sync_copy(zrow_hbm, acc_sh.at[pl.ds(base, RPS)])

    @pl.when(last)
    def _():
        pltpu.sync_copy(zrow_hbm.at[pl.ds(0, TAIL)], acc_sh.at[pl.ds(NS * RPS, TAIL)])

    if with_deg:
        pltpu.sync_copy(ones_hbm, ones_v)
        pltpu.sync_copy(zdeg_hbm, deg_sh.at[pl.ds(base, RPS)])

        @pl.when(last)
        def _():
            pltpu.sync_copy(zdeg_hbm.at[pl.ds(0, TAIL)],
                            deg_sh.at[pl.ds(NS * RPS, TAIL)])

    plsc.subcore_barrier()

    def giter(g, carry):
        jbase = g * NBUF
        for b in range(NBUF):
            j = jbase + b
            pltpu.make_async_copy(table_hbm.at[idx_v.at[j]],
                                  rows_v.at[b], gsem).wait()
            pltpu.async_copy(rows_v.at[b], acc_sh.at[dst_v.at[j]], ssem, add=True)
            if with_deg:
                @pl.when(lax.rem(j, NC) == cid)
                def _():
                    pltpu.async_copy(ones_v, deg_sh.at[dst_v.at[j]], dsem,
                                     add=True)
        for b in range(NBUF):
            j = jbase + b
            pltpu.make_async_copy(rows_v.at[b], acc_sh.at[dst_v.at[j]], ssem).wait()

            @pl.when(g + 1 < NITER)
            def _():
                pltpu.async_copy(table_hbm.at[idx_v.at[j + NBUF]], rows_v.at[b],
                                 gsem)
        return carry

    lax.fori_loop(0, NITER, giter, 0)

    if with_deg:
        def ddrain(i, carry):
            pltpu.make_async_copy(ones_v, deg_sh.at[dst_v.at[0]], dsem).wait()
            return carry

        lax.fori_loop(0, NCHUNK // NC, ddrain, 0)

    plsc.subcore_barrier()

    pltpu.sync_copy(acc_sh.at[pl.ds(base, RPS)], sum_hbm.at[cid, pl.ds(base, RPS)])

    @pl.when(last)
    def _():
        pltpu.sync_copy(acc_sh.at[pl.ds(NS * RPS, TAIL)],
                        sum_hbm.at[cid, pl.ds(NS * RPS, TAIL)])

    if with_deg:
        pltpu.sync_copy(deg_sh.at[pl.ds(base, RPS)], deg_hbm.at[cid, pl.ds(base, RPS)])

        @pl.when(last)
        def _():
            pltpu.sync_copy(deg_sh.at[pl.ds(NS * RPS, TAIL)],
                            deg_hbm.at[cid, pl.ds(NS * RPS, TAIL)])


def _make_sc_scatter(with_deg):
    out_type = [jax.ShapeDtypeStruct((NC, N, H), jnp.float32)]
    if with_deg:
        out_type.append(jax.ShapeDtypeStruct((NC, N, 16), jnp.float32))
    scratch = [
        pltpu.VMEM((NCHUNK, C), jnp.int32),
        pltpu.VMEM((NCHUNK, C), jnp.int32),
        pltpu.VMEM((NBUF, C, H), jnp.float32),
    ]
    if with_deg:
        scratch.append(pltpu.VMEM((C, 16), jnp.float32))
    scratch.append(pltpu.VMEM_SHARED((N, H), jnp.float32))
    if with_deg:
        scratch.append(pltpu.VMEM_SHARED((N, 16), jnp.float32))
    scratch.append(pltpu.SemaphoreType.DMA)
    scratch.append(pltpu.SemaphoreType.DMA)
    if with_deg:
        scratch.append(pltpu.SemaphoreType.DMA)

    mesh = plsc.VectorSubcoreMesh(core_axis_name="c", subcore_axis_name="s")
    return pl.kernel(
        functools.partial(_sc_body, with_deg),
        out_type=out_type,
        mesh=mesh,
        scratch_types=scratch,
        compiler_params=pltpu.CompilerParams(use_tc_tiling_on_sc=False),
    )


_sc_scatter_deg = _make_sc_scatter(True)
_sc_scatter = _make_sc_scatter(False)


def _tc_layer_body(relu, s_ref, d_ref, x_ref, wl_ref, wr_ref, b_ref, out_ref):
    deg = jnp.maximum(d_ref[0, :, 0:1] + d_ref[1, :, 0:1], 1.0)
    inv = 1.0 / deg
    z = (jnp.dot(s_ref[0] * inv, wl_ref[0:H, :], preferred_element_type=jnp.float32)
         + jnp.dot(s_ref[1] * inv, wl_ref[H:D, :], preferred_element_type=jnp.float32)
         + jnp.dot(x_ref[...], wr_ref[...], preferred_element_type=jnp.float32)
         + b_ref[...])
    out_ref[...] = jnp.maximum(z, 0.0) if relu else z


_TC_BLOCK = 1000


def _tc_layer(relu, s, d, x, wl_t, wr_t, b):
    grid = (N // _TC_BLOCK,)
    return pl.pallas_call(
        functools.partial(_tc_layer_body, relu),
        grid=grid,
        in_specs=[
            pl.BlockSpec((NC, _TC_BLOCK, H), lambda i: (0, i, 0)),
            pl.BlockSpec((NC, _TC_BLOCK, 16), lambda i: (0, i, 0)),
            pl.BlockSpec((_TC_BLOCK, D), lambda i: (i, 0)),
            pl.BlockSpec((D, D), lambda i: (0, 0)),
            pl.BlockSpec((D, D), lambda i: (0, 0)),
            pl.BlockSpec((1, D), lambda i: (0, 0)),
        ],
        out_specs=pl.BlockSpec((_TC_BLOCK, D), lambda i: (i, 0)),
        out_shape=jax.ShapeDtypeStruct((N, D), jnp.float32),
    )(s, d, x, wl_t, wr_t, b)


def kernel(x, edge_index, W1_l, W1_r, b1, W2_l, W2_r, b2):
    src = edge_index[0].astype(jnp.int32).reshape(NS, NCHUNK, C)
    dst = edge_index[1].astype(jnp.int32).reshape(NS, NCHUNK, C)
    src2 = jnp.concatenate([2 * src, 2 * src + 1], axis=0)
    zrow = jnp.zeros((RPS, H), jnp.float32)
    zdeg = jnp.zeros((RPS, 16), jnp.float32)
    ones = jnp.ones((C, 16), jnp.float32)

    s1, deg = _sc_scatter_deg(x.reshape(2 * N, H), src2, dst, zrow, zdeg, ones)
    h = _tc_layer(True, s1, deg, x, W1_l.T, W1_r.T, b1.reshape(1, D))
    (s2,) = _sc_scatter(h.reshape(2 * N, H), src2, dst, zrow)
    out = _tc_layer(False, s2, deg, h, W2_l.T, W2_r.T, b2.reshape(1, D))
    return out


# speedup vs baseline: 1.1252x; 1.0173x over previous
import functools

import jax
import jax.numpy as jnp
from jax import lax
from jax.experimental import pallas as pl
from jax.experimental.pallas import tpu as pltpu
from jax.experimental.pallas import tpu_sc as plsc

N = 10000
E = 320000
D = 128
H = D // 2
NC = 2
NS = 16
EPW = E // NS
C = 80
NCHUNK = EPW // C
NBUF = 5
LAG = NBUF // 2
NITER = NCHUNK // NBUF
RPS = 624
TAIL = N - NS * RPS


def _sc_body(with_deg, *refs):
    if with_deg:
        (table_hbm, src_hbm, dst_hbm, zrow_hbm, zdeg_hbm, ones_hbm,
         sum_hbm, deg_hbm,
         idx_v, dst_v, rows_v, ones_v, acc_sh, deg_sh, gsem, ssem, dsem) = refs
    else:
        (table_hbm, src_hbm, dst_hbm, zrow_hbm,
         sum_hbm,
         idx_v, dst_v, rows_v, acc_sh, gsem, ssem) = refs

    cid = lax.axis_index("c")
    sid = lax.axis_index("s")
    last = sid == NS - 1

    pltpu.sync_copy(src_hbm.at[cid * NS + sid], idx_v)
    pltpu.sync_copy(dst_hbm.at[sid], dst_v)
    for b in range(NBUF):
        pltpu.async_copy(table_hbm.at[idx_v.at[b]], rows_v.at[b], gsem)

    base = pl.multiple_of(sid * RPS, 8)
    pltpu.sync_copy(zrow_hbm, acc_sh.at[pl.ds(base, RPS)])

    @pl.when(last)
    def _():
        pltpu.sync_copy(zrow_hbm.at[pl.ds(0, TAIL)], acc_sh.at[pl.ds(NS * RPS, TAIL)])

    if with_deg:
        pltpu.sync_copy(ones_hbm, ones_v)
        pltpu.sync_copy(zdeg_hbm, deg_sh.at[pl.ds(base, RPS)])

        @pl.when(last)
        def _():
            pltpu.sync_copy(zdeg_hbm.at[pl.ds(0, TAIL)],
                            deg_sh.at[pl.ds(NS * RPS, TAIL)])

    plsc.subcore_barrier()

    def giter(g, carry):
        jbase = g * NBUF
        for b in range(NBUF):
            j = jbase + b
            pltpu.make_async_copy(table_hbm.at[idx_v.at[j]],
                                  rows_v.at[b], gsem).wait()
            pltpu.async_copy(rows_v.at[b], acc_sh.at[dst_v.at[j]], ssem, add=True)
            if with_deg:
                @pl.when(lax.rem(j, NC) == cid)
                def _():
                    pltpu.async_copy(ones_v, deg_sh.at[dst_v.at[j]], dsem,
                                     add=True)
        for b in range(NBUF):
            j = jbase + b
            pltpu.make_async_copy(rows_v.at[b], acc_sh.at[dst_v.at[j]], ssem).wait()

            @pl.when(g + 1 < NITER)
            def _():
                pltpu.async_copy(table_hbm.at[idx_v.at[j + NBUF]], rows_v.at[b],
                                 gsem)
        return carry

    lax.fori_loop(0, NITER, giter, 0)

    if with_deg:
        def ddrain(i, carry):
            pltpu.make_async_copy(ones_v, deg_sh.at[dst_v.at[0]], dsem).wait()
            return carry

        lax.fori_loop(0, NCHUNK // NC, ddrain, 0)

    plsc.subcore_barrier()

    pltpu.sync_copy(acc_sh.at[pl.ds(base, RPS)], sum_hbm.at[cid, pl.ds(base, RPS)])

    @pl.when(last)
    def _():
        pltpu.sync_copy(acc_sh.at[pl.ds(NS * RPS, TAIL)],
                        sum_hbm.at[cid, pl.ds(NS * RPS, TAIL)])

    if with_deg:
        pltpu.sync_copy(deg_sh.at[pl.ds(base, RPS)], deg_hbm.at[cid, pl.ds(base, RPS)])

        @pl.when(last)
        def _():
            pltpu.sync_copy(deg_sh.at[pl.ds(NS * RPS, TAIL)],
                            deg_hbm.at[cid, pl.ds(NS * RPS, TAIL)])


def _make_sc_scatter(with_deg):
    out_type = [jax.ShapeDtypeStruct((NC, N, H), jnp.float32)]
    if with_deg:
        out_type.append(jax.ShapeDtypeStruct((NC, N, 16), jnp.float32))
    scratch = [
        pltpu.VMEM((NCHUNK, C), jnp.int32),
        pltpu.VMEM((NCHUNK, C), jnp.int32),
        pltpu.VMEM((NBUF, C, H), jnp.float32),
    ]
    if with_deg:
        scratch.append(pltpu.VMEM((C, 16), jnp.float32))
    scratch.append(pltpu.VMEM_SHARED((N, H), jnp.float32))
    if with_deg:
        scratch.append(pltpu.VMEM_SHARED((N, 16), jnp.float32))
    scratch.append(pltpu.SemaphoreType.DMA)
    scratch.append(pltpu.SemaphoreType.DMA)
    if with_deg:
        scratch.append(pltpu.SemaphoreType.DMA)

    mesh = plsc.VectorSubcoreMesh(core_axis_name="c", subcore_axis_name="s")
    return pl.kernel(
        functools.partial(_sc_body, with_deg),
        out_type=out_type,
        mesh=mesh,
        scratch_types=scratch,
        compiler_params=pltpu.CompilerParams(use_tc_tiling_on_sc=False),
    )


_sc_scatter_deg = _make_sc_scatter(True)
_sc_scatter = _make_sc_scatter(False)


def _tc_layer_body(relu, s_ref, d_ref, x_ref, wl_ref, wr_ref, b_ref, out_ref):
    deg = jnp.maximum(d_ref[0, :, 0:1] + d_ref[1, :, 0:1], 1.0)
    inv = 1.0 / deg
    z = (jnp.dot(s_ref[0] * inv, wl_ref[0:H, :], preferred_element_type=jnp.float32)
         + jnp.dot(s_ref[1] * inv, wl_ref[H:D, :], preferred_element_type=jnp.float32)
         + jnp.dot(x_ref[...], wr_ref[...], preferred_element_type=jnp.float32)
         + b_ref[...])
    out_ref[...] = jnp.maximum(z, 0.0) if relu else z


_TC_BLOCK = 2000


def _tc_layer(relu, s, d, x, wl_t, wr_t, b):
    grid = (N // _TC_BLOCK,)
    return pl.pallas_call(
        functools.partial(_tc_layer_body, relu),
        grid=grid,
        in_specs=[
            pl.BlockSpec((NC, _TC_BLOCK, H), lambda i: (0, i, 0)),
            pl.BlockSpec((NC, _TC_BLOCK, 16), lambda i: (0, i, 0)),
            pl.BlockSpec((_TC_BLOCK, D), lambda i: (i, 0)),
            pl.BlockSpec((D, D), lambda i: (0, 0)),
            pl.BlockSpec((D, D), lambda i: (0, 0)),
            pl.BlockSpec((1, D), lambda i: (0, 0)),
        ],
        out_specs=pl.BlockSpec((_TC_BLOCK, D), lambda i: (i, 0)),
        out_shape=jax.ShapeDtypeStruct((N, D), jnp.float32),
    )(s, d, x, wl_t, wr_t, b)


def kernel(x, edge_index, W1_l, W1_r, b1, W2_l, W2_r, b2):
    src = edge_index[0].astype(jnp.int32).reshape(NS, NCHUNK, C)
    dst = edge_index[1].astype(jnp.int32).reshape(NS, NCHUNK, C)
    src2 = jnp.concatenate([2 * src, 2 * src + 1], axis=0)
    zrow = jnp.zeros((RPS, H), jnp.float32)
    zdeg = jnp.zeros((RPS, 16), jnp.float32)
    ones = jnp.ones((C, 16), jnp.float32)

    s1, deg = _sc_scatter_deg(x.reshape(2 * N, H), src2, dst, zrow, zdeg, ones)
    h = _tc_layer(True, s1, deg, x, W1_l.T, W1_r.T, b1.reshape(1, D))
    (s2,) = _sc_scatter(h.reshape(2 * N, H), src2, dst, zrow)
    out = _tc_layer(False, s2, deg, h, W2_l.T, W2_r.T, b2.reshape(1, D))
    return out


# speedup vs baseline: 1.1622x; 1.0329x over previous
import functools

import jax
import jax.numpy as jnp
from jax import lax
from jax.experimental import pallas as pl
from jax.experimental.pallas import tpu as pltpu
from jax.experimental.pallas import tpu_sc as plsc

N = 10000
E = 320000
D = 128
H = D // 2
NC = 2
NS = 16
EPW = E // NS
C = 80
NCHUNK = EPW // C
NBUF = 5
LAG = NBUF // 2
NITER = NCHUNK // NBUF
RPS = 624
TAIL = N - NS * RPS


def _sc_body(with_deg, *refs):
    if with_deg:
        (table_hbm, src_hbm, dst_hbm, zrow_hbm, zdeg_hbm, ones_hbm,
         sum_hbm, deg_hbm,
         idx_v, dst_v, rows_v, ones_v, acc_sh, deg_sh, gsem, ssem, dsem) = refs
    else:
        (table_hbm, src_hbm, dst_hbm, zrow_hbm,
         sum_hbm,
         idx_v, dst_v, rows_v, acc_sh, gsem, ssem) = refs

    cid = lax.axis_index("c")
    sid = lax.axis_index("s")
    last = sid == NS - 1

    table_c = table_hbm.at[pl.ds(cid, 2 * N - 1)]

    pltpu.sync_copy(src_hbm.at[sid], idx_v)
    pltpu.sync_copy(dst_hbm.at[sid], dst_v)
    for b in range(NBUF):
        pltpu.async_copy(table_c.at[idx_v.at[b]], rows_v.at[b], gsem)

    base = pl.multiple_of(sid * RPS, 8)
    pltpu.sync_copy(zrow_hbm, acc_sh.at[pl.ds(base, RPS)])

    @pl.when(last)
    def _():
        pltpu.sync_copy(zrow_hbm.at[pl.ds(0, TAIL)], acc_sh.at[pl.ds(NS * RPS, TAIL)])

    if with_deg:
        pltpu.sync_copy(ones_hbm, ones_v)
        pltpu.sync_copy(zdeg_hbm, deg_sh.at[pl.ds(base, RPS)])

        @pl.when(last)
        def _():
            pltpu.sync_copy(zdeg_hbm.at[pl.ds(0, TAIL)],
                            deg_sh.at[pl.ds(NS * RPS, TAIL)])

    plsc.subcore_barrier()

    def giter(g, carry):
        jbase = g * NBUF
        for b in range(NBUF):
            j = jbase + b
            pltpu.make_async_copy(table_c.at[idx_v.at[j]],
                                  rows_v.at[b], gsem).wait()
            pltpu.async_copy(rows_v.at[b], acc_sh.at[dst_v.at[j]], ssem, add=True)
            if with_deg:
                @pl.when(lax.rem(j, NC) == cid)
                def _():
                    pltpu.async_copy(ones_v, deg_sh.at[dst_v.at[j]], dsem,
                                     add=True)
        for b in range(NBUF):
            j = jbase + b
            pltpu.make_async_copy(rows_v.at[b], acc_sh.at[dst_v.at[j]], ssem).wait()

            @pl.when(g + 1 < NITER)
            def _():
                pltpu.async_copy(table_c.at[idx_v.at[j + NBUF]], rows_v.at[b],
                                 gsem)
        return carry

    lax.fori_loop(0, NITER, giter, 0)

    if with_deg:
        def ddrain(i, carry):
            pltpu.make_async_copy(ones_v, deg_sh.at[dst_v.at[0]], dsem).wait()
            return carry

        lax.fori_loop(0, NCHUNK // NC, ddrain, 0)

    plsc.subcore_barrier()

    pltpu.sync_copy(acc_sh.at[pl.ds(base, RPS)], sum_hbm.at[cid, pl.ds(base, RPS)])

    @pl.when(last)
    def _():
        pltpu.sync_copy(acc_sh.at[pl.ds(NS * RPS, TAIL)],
                        sum_hbm.at[cid, pl.ds(NS * RPS, TAIL)])

    if with_deg:
        pltpu.sync_copy(deg_sh.at[pl.ds(base, RPS)], deg_hbm.at[cid, pl.ds(base, RPS)])

        @pl.when(last)
        def _():
            pltpu.sync_copy(deg_sh.at[pl.ds(NS * RPS, TAIL)],
                            deg_hbm.at[cid, pl.ds(NS * RPS, TAIL)])


def _make_sc_scatter(with_deg):
    out_type = [jax.ShapeDtypeStruct((NC, N, H), jnp.float32)]
    if with_deg:
        out_type.append(jax.ShapeDtypeStruct((NC, N, 16), jnp.float32))
    scratch = [
        pltpu.VMEM((NCHUNK, C), jnp.int32),
        pltpu.VMEM((NCHUNK, C), jnp.int32),
        pltpu.VMEM((NBUF, C, H), jnp.float32),
    ]
    if with_deg:
        scratch.append(pltpu.VMEM((C, 16), jnp.float32))
    scratch.append(pltpu.VMEM_SHARED((N, H), jnp.float32))
    if with_deg:
        scratch.append(pltpu.VMEM_SHARED((N, 16), jnp.float32))
    scratch.append(pltpu.SemaphoreType.DMA)
    scratch.append(pltpu.SemaphoreType.DMA)
    if with_deg:
        scratch.append(pltpu.SemaphoreType.DMA)

    mesh = plsc.VectorSubcoreMesh(core_axis_name="c", subcore_axis_name="s")
    return pl.kernel(
        functools.partial(_sc_body, with_deg),
        out_type=out_type,
        mesh=mesh,
        scratch_types=scratch,
        compiler_params=pltpu.CompilerParams(use_tc_tiling_on_sc=False),
    )


_sc_scatter_deg = _make_sc_scatter(True)
_sc_scatter = _make_sc_scatter(False)


def _tc_layer_body(relu, s_ref, d_ref, x_ref, wl_ref, wr_ref, b_ref, out_ref):
    deg = jnp.maximum(d_ref[0, :, 0:1] + d_ref[1, :, 0:1], 1.0)
    inv = 1.0 / deg
    z = (jnp.dot(s_ref[0] * inv, wl_ref[0:H, :], preferred_element_type=jnp.float32)
         + jnp.dot(s_ref[1] * inv, wl_ref[H:D, :], preferred_element_type=jnp.float32)
         + jnp.dot(x_ref[...], wr_ref[...], preferred_element_type=jnp.float32)
         + b_ref[...])
    out_ref[...] = jnp.maximum(z, 0.0) if relu else z


_TC_BLOCK = 2000


def _tc_layer(relu, s, d, x, wl_t, wr_t, b):
    grid = (N // _TC_BLOCK,)
    return pl.pallas_call(
        functools.partial(_tc_layer_body, relu),
        grid=grid,
        in_specs=[
            pl.BlockSpec((NC, _TC_BLOCK, H), lambda i: (0, i, 0)),
            pl.BlockSpec((NC, _TC_BLOCK, 16), lambda i: (0, i, 0)),
            pl.BlockSpec((_TC_BLOCK, D), lambda i: (i, 0)),
            pl.BlockSpec((D, D), lambda i: (0, 0)),
            pl.BlockSpec((D, D), lambda i: (0, 0)),
            pl.BlockSpec((1, D), lambda i: (0, 0)),
        ],
        out_specs=pl.BlockSpec((_TC_BLOCK, D), lambda i: (i, 0)),
        out_shape=jax.ShapeDtypeStruct((N, D), jnp.float32),
    )(s, d, x, wl_t, wr_t, b)


def kernel(x, edge_index, W1_l, W1_r, b1, W2_l, W2_r, b2):
    src2 = (edge_index[0].astype(jnp.int32) * 2).reshape(NS, NCHUNK, C)
    dst = edge_index[1].astype(jnp.int32).reshape(NS, NCHUNK, C)
    zrow = jnp.zeros((RPS, H), jnp.float32)
    zdeg = jnp.zeros((RPS, 16), jnp.float32)
    ones = jnp.ones((C, 16), jnp.float32)

    s1, deg = _sc_scatter_deg(x.reshape(2 * N, H), src2, dst, zrow, zdeg, ones)
    h = _tc_layer(True, s1, deg, x, W1_l.T, W1_r.T, b1.reshape(1, D))
    (s2,) = _sc_scatter(h.reshape(2 * N, H), src2, dst, zrow)
    out = _tc_layer(False, s2, deg, h, W2_l.T, W2_r.T, b2.reshape(1, D))
    return out


# speedup vs baseline: 1.2144x; 1.0449x over previous
import functools

import jax
import jax.numpy as jnp
from jax import lax
from jax.experimental import pallas as pl
from jax.experimental.pallas import tpu as pltpu
from jax.experimental.pallas import tpu_sc as plsc

N = 10000
E = 320000
D = 128
H = D // 2
NC = 2
NS = 16
EPW = E // NS
C = 80
NCHUNK = EPW // C
GRP = 5
GB = GRP * C
RING = 2 * GRP
W = 50
NWIN = NCHUNK // W
GPW = W // GRP
RPS = 624
TAIL = N - NS * RPS


def _sc_body(with_deg, *refs):
    if with_deg:
        (table_hbm, src_hbm, dst_hbm, zrow_hbm, zdeg_hbm, ones_hbm,
         sum_hbm, deg_hbm,
         idx_v, dst_v, rows_v, ones_v, acc_sh, deg_sh, gsem, ssem, isem,
         dsem) = refs
    else:
        (table_hbm, src_hbm, dst_hbm, zrow_hbm,
         sum_hbm,
         idx_v, dst_v, rows_v, acc_sh, gsem, ssem, isem) = refs

    cid = lax.axis_index("c")
    sid = lax.axis_index("s")
    last = sid == NS - 1

    table_c = table_hbm.at[pl.ds(cid, 2 * N - 1)]

    pltpu.sync_copy(src_hbm.at[sid, pl.ds(0, W)], idx_v.at[0])
    pltpu.sync_copy(dst_hbm.at[sid, pl.ds(0, W)], dst_v.at[0])
    for b in range(RING):
        pltpu.async_copy(table_c.at[idx_v.at[0, b]],
                         rows_v.at[pl.ds(b * C, C)], gsem)

    base = pl.multiple_of(sid * RPS, 8)
    pltpu.sync_copy(zrow_hbm, acc_sh.at[pl.ds(base, RPS)])

    @pl.when(last)
    def _():
        pltpu.sync_copy(zrow_hbm.at[pl.ds(0, TAIL)], acc_sh.at[pl.ds(NS * RPS, TAIL)])

    if with_deg:
        pltpu.sync_copy(ones_hbm, ones_v)
        pltpu.sync_copy(zdeg_hbm, deg_sh.at[pl.ds(base, RPS)])

        @pl.when(last)
        def _():
            pltpu.sync_copy(zdeg_hbm.at[pl.ds(0, TAIL)],
                            deg_sh.at[pl.ds(NS * RPS, TAIL)])

    plsc.subcore_barrier()

    def group_body(w, pw, g):
        G = w * GPW + g
        sb = lax.rem(G, 2) * GB
        jbase = w * W + g * GRP
        lbase = g * GRP
        pltpu.make_async_copy(table_hbm.at[pl.ds(0, GB)],
                              rows_v.at[pl.ds(0, GB)], gsem).wait()
        for b in range(GRP):
            pltpu.async_copy(rows_v.at[pl.ds(sb + b * C, C)],
                             acc_sh.at[dst_v.at[pw, lbase + b]], ssem, add=True)
            if with_deg:
                @pl.when(lax.rem(jbase + b, NC) == cid)
                def _():
                    pltpu.async_copy(ones_v, deg_sh.at[dst_v.at[pw, lbase + b]],
                                     dsem, add=True)
        pltpu.make_async_copy(rows_v.at[pl.ds(0, GB)],
                              acc_sh.at[pl.ds(0, GB)], ssem).wait()
        for b in range(GRP):
            j2 = jbase + 2 * GRP + b
            l2 = lbase + 2 * GRP + b
            qp = lax.rem(pw + l2 // W, 2)
            r2 = lax.rem(l2, W)

            @pl.when(j2 < NCHUNK)
            def _():
                pltpu.async_copy(table_c.at[idx_v.at[qp, r2]],
                                 rows_v.at[pl.ds(sb + b * C, C)], gsem)

    def wloop(w, carry):
        pw = lax.rem(w, 2)

        @pl.when(w + 1 < NWIN)
        def _():
            pn = lax.rem(w + 1, 2)
            pltpu.async_copy(src_hbm.at[sid, pl.ds((w + 1) * W, W)],
                             idx_v.at[pn], isem)
            pltpu.async_copy(dst_hbm.at[sid, pl.ds((w + 1) * W, W)],
                             dst_v.at[pn], isem)

        def g_early(g, c2):
            group_body(w, pw, g)
            return c2

        lax.fori_loop(0, GPW - 2, g_early, 0)

        @pl.when(w + 1 < NWIN)
        def _():
            pltpu.make_async_copy(src_hbm.at[sid, pl.ds(0, W)],
                                  idx_v.at[0], isem).wait()
            pltpu.make_async_copy(dst_hbm.at[sid, pl.ds(0, W)],
                                  dst_v.at[0], isem).wait()

        def g_late(g, c2):
            group_body(w, pw, g)
            return c2

        lax.fori_loop(GPW - 2, GPW, g_late, 0)
        return carry

    lax.fori_loop(0, NWIN, wloop, 0)

    if with_deg:
        def ddrain(i, carry):
            pltpu.make_async_copy(ones_v, deg_sh.at[dst_v.at[0]], dsem).wait()
            return carry

        lax.fori_loop(0, NCHUNK // NC, ddrain, 0)

    plsc.subcore_barrier()

    pltpu.sync_copy(acc_sh.at[pl.ds(base, RPS)], sum_hbm.at[cid, pl.ds(base, RPS)])

    @pl.when(last)
    def _():
        pltpu.sync_copy(acc_sh.at[pl.ds(NS * RPS, TAIL)],
                        sum_hbm.at[cid, pl.ds(NS * RPS, TAIL)])

    if with_deg:
        pltpu.sync_copy(deg_sh.at[pl.ds(base, RPS)], deg_hbm.at[cid, pl.ds(base, RPS)])

        @pl.when(last)
        def _():
            pltpu.sync_copy(deg_sh.at[pl.ds(NS * RPS, TAIL)],
                            deg_hbm.at[cid, pl.ds(NS * RPS, TAIL)])


def _make_sc_scatter(with_deg):
    out_type = [jax.ShapeDtypeStruct((NC, N, H), jnp.float32)]
    if with_deg:
        out_type.append(jax.ShapeDtypeStruct((NC, N, 16), jnp.float32))
    scratch = [
        pltpu.VMEM((2, W, C), jnp.int32),
        pltpu.VMEM((2, W, C), jnp.int32),
        pltpu.VMEM((RING * C, H), jnp.float32),
    ]
    if with_deg:
        scratch.append(pltpu.VMEM((C, 16), jnp.float32))
    scratch.append(pltpu.VMEM_SHARED((N, H), jnp.float32))
    if with_deg:
        scratch.append(pltpu.VMEM_SHARED((N, 16), jnp.float32))
    scratch.append(pltpu.SemaphoreType.DMA)
    scratch.append(pltpu.SemaphoreType.DMA)
    scratch.append(pltpu.SemaphoreType.DMA)
    if with_deg:
        scratch.append(pltpu.SemaphoreType.DMA)

    mesh = plsc.VectorSubcoreMesh(core_axis_name="c", subcore_axis_name="s")
    return pl.kernel(
        functools.partial(_sc_body, with_deg),
        out_type=out_type,
        mesh=mesh,
        scratch_types=scratch,
        compiler_params=pltpu.CompilerParams(use_tc_tiling_on_sc=False),
    )


_sc_scatter_deg = _make_sc_scatter(True)
_sc_scatter = _make_sc_scatter(False)


def _tc_layer_body(relu, s_ref, d_ref, x_ref, wl_ref, wr_ref, b_ref, out_ref):
    deg = jnp.maximum(d_ref[0, :, 0:1] + d_ref[1, :, 0:1], 1.0)
    inv = 1.0 / deg
    z = (jnp.dot(s_ref[0] * inv, wl_ref[0:H, :], preferred_element_type=jnp.float32)
         + jnp.dot(s_ref[1] * inv, wl_ref[H:D, :], preferred_element_type=jnp.float32)
         + jnp.dot(x_ref[...], wr_ref[...], preferred_element_type=jnp.float32)
         + b_ref[...])
    out_ref[...] = jnp.maximum(z, 0.0) if relu else z


_TC_BLOCK = 2000


def _tc_layer(relu, s, d, x, wl_t, wr_t, b):
    grid = (N // _TC_BLOCK,)
    return pl.pallas_call(
        functools.partial(_tc_layer_body, relu),
        grid=grid,
        in_specs=[
            pl.BlockSpec((NC, _TC_BLOCK, H), lambda i: (0, i, 0)),
            pl.BlockSpec((NC, _TC_BLOCK, 16), lambda i: (0, i, 0)),
            pl.BlockSpec((_TC_BLOCK, D), lambda i: (i, 0)),
            pl.BlockSpec((D, D), lambda i: (0, 0)),
            pl.BlockSpec((D, D), lambda i: (0, 0)),
            pl.BlockSpec((1, D), lambda i: (0, 0)),
        ],
        out_specs=pl.BlockSpec((_TC_BLOCK, D), lambda i: (i, 0)),
        out_shape=jax.ShapeDtypeStruct((N, D), jnp.float32),
    )(s, d, x, wl_t, wr_t, b)


def kernel(x, edge_index, W1_l, W1_r, b1, W2_l, W2_r, b2):
    src2 = (edge_index[0].astype(jnp.int32) * 2).reshape(NS, NCHUNK, C)
    dst = edge_index[1].astype(jnp.int32).reshape(NS, NCHUNK, C)
    zrow = jnp.zeros((RPS, H), jnp.float32)
    zdeg = jnp.zeros((RPS, 16), jnp.float32)
    ones = jnp.ones((C, 16), jnp.float32)

    s1, deg = _sc_scatter_deg(x.reshape(2 * N, H), src2, dst, zrow, zdeg, ones)
    h = _tc_layer(True, s1, deg, x, W1_l.T, W1_r.T, b1.reshape(1, D))
    (s2,) = _sc_scatter(h.reshape(2 * N, H), src2, dst, zrow)
    out = _tc_layer(False, s2, deg, h, W2_l.T, W2_r.T, b2.reshape(1, D))
    return out
